# Initial kernel scaffold; baseline (speedup 1.0000x reference)
import jax
import jax.numpy as jnp
from jax.experimental import pallas as pl


def kernel(diag_codes, proc_codes, prev_diag_codes, prev_proc_codes, prev_med_codes, diag_table, proc_table, med_table, W1, b1, W2, b2, W3, b3):
    raise NotImplementedError("write your pallas kernel here")


# speedup vs baseline: 2.6239x; 2.6239x over previous
import functools

import jax
import jax.numpy as jnp
from jax import lax
from jax.experimental import pallas as pl
from jax.experimental.pallas import tpu as pltpu
from jax.experimental.pallas import tpu_sc as plsc

B = 1024
L = 50
D = 128
NLOOK = 5
NC, NS = 2, 16
NW = NC * NS
BPW = B // NW
CHUNK_ROWS = 2
IDX_PER_CHUNK = CHUNK_ROWS * L
NCHUNK = BPW // CHUNK_ROWS
DCH = D // 16


def _sc_pool_body(codes_hbm, diag_t, proc_t, med_t, out_hbm,
                  idx_v, rows_v, acc_v, sem):
    wid = lax.axis_index("s") * NC + lax.axis_index("c")
    base = wid * BPW
    tables = (diag_t, proc_t, diag_t, proc_t, med_t)
    for look in range(NLOOK):
        table = tables[look]
        pltpu.sync_copy(codes_hbm.at[look, wid], idx_v)

        def chunk_body(g, carry, table=table):
            pltpu.async_copy(table.at[idx_v.at[g]], rows_v, sem).wait()

            def red(j, accs):
                a0 = tuple(accs[d] + rows_v[j, pl.ds(d * 16, 16)]
                           for d in range(DCH))
                a1 = tuple(accs[DCH + d] + rows_v[L + j, pl.ds(d * 16, 16)]
                           for d in range(DCH))
                return a0 + a1

            zero = jnp.zeros((16,), jnp.float32)
            accs = lax.fori_loop(0, L, red, (zero,) * (2 * DCH))
            for d in range(DCH):
                acc_v[2 * g, pl.ds(d * 16, 16)] = accs[d]
                acc_v[2 * g + 1, pl.ds(d * 16, 16)] = accs[DCH + d]
            return carry

        lax.fori_loop(0, NCHUNK, chunk_body, 0)
        pltpu.sync_copy(acc_v, out_hbm.at[look, pl.ds(base, BPW)])


def _sc_pool(codes, diag_table, proc_table, med_table):
    run = functools.partial(
        pl.kernel,
        mesh=plsc.VectorSubcoreMesh(core_axis_name="c", subcore_axis_name="s"),
        out_type=jax.ShapeDtypeStruct((NLOOK, B, D), jnp.float32),
        scratch_types=[
            pltpu.VMEM((NCHUNK, IDX_PER_CHUNK), jnp.int32),
            pltpu.VMEM((IDX_PER_CHUNK, D), jnp.float32),
            pltpu.VMEM((BPW, D), jnp.float32),
            pltpu.SemaphoreType.DMA,
        ],
    )(_sc_pool_body)
    return run(codes, diag_table, proc_table, med_table)


OUT_RAW = 1000
OUT_PAD = 1024
BLK = 128


def _mlp_body(ed, ep, epd, epp, em, w1a, w1b, b1r, w2a, w2b, w2c, b2r,
              w3t, b3r, out):
    cur = ed[:] @ w1a[:] + ep[:] @ w1b[:] + b1r[:]
    prev = epd[:] @ w1a[:] + epp[:] @ w1b[:] + b1r[:]
    h = jnp.maximum(
        cur @ w2a[:] + prev @ w2b[:] + em[:] @ w2c[:] + b2r[:], 0.0)
    out[:] = jax.nn.sigmoid(h @ w3t[:] + b3r[:])


def _mlp(ed, ep, epd, epp, em, w1a, w1b, b1r, w2a, w2b, w2c, b2r, w3t, b3r):
    full = lambda shape: pl.BlockSpec(shape, lambda i: (0, 0))
    blk = pl.BlockSpec((BLK, D), lambda i: (i, 0))
    return pl.pallas_call(
        _mlp_body,
        grid=(B // BLK,),
        in_specs=[blk, blk, blk, blk, blk,
                  full((D, D)), full((D, D)), full((1, D)),
                  full((D, 256)), full((D, 256)), full((D, 256)),
                  full((1, 256)),
                  full((256, OUT_PAD)), full((1, OUT_PAD))],
        out_specs=pl.BlockSpec((BLK, OUT_PAD), lambda i: (i, 0)),
        out_shape=jax.ShapeDtypeStruct((B, OUT_PAD), jnp.float32),
    )(ed, ep, epd, epp, em, w1a, w1b, b1r, w2a, w2b, w2c, b2r, w3t, b3r)


def kernel(diag_codes, proc_codes, prev_diag_codes, prev_proc_codes,
           prev_med_codes, diag_table, proc_table, med_table,
           W1, b1, W2, b2, W3, b3):
    codes = jnp.stack([diag_codes, proc_codes, prev_diag_codes,
                       prev_proc_codes, prev_med_codes])
    codes = codes.astype(jnp.int32).reshape(NLOOK, NW, NCHUNK, IDX_PER_CHUNK)
    pooled = _sc_pool(codes, diag_table, proc_table, med_table)

    w1a = W1[:, :D].T
    w1b = W1[:, D:].T
    w2a = W2[:, :D].T
    w2b = W2[:, D:2 * D].T
    w2c = W2[:, 2 * D:].T
    w3t = jnp.zeros((256, OUT_PAD), jnp.float32).at[:, :OUT_RAW].set(W3.T)
    b3r = jnp.zeros((1, OUT_PAD), jnp.float32).at[0, :OUT_RAW].set(b3)
    out = _mlp(pooled[0], pooled[1], pooled[2], pooled[3], pooled[4],
               w1a, w1b, b1.reshape(1, D), w2a, w2b, w2c, b2.reshape(1, 256),
               w3t, b3r)
    return out[:, :OUT_RAW]


# speedup vs baseline: 3.7915x; 1.4450x over previous
import functools

import jax
import jax.numpy as jnp
from jax import lax
from jax.experimental import pallas as pl
from jax.experimental.pallas import tpu as pltpu
from jax.experimental.pallas import tpu_sc as plsc

B = 1024
L = 50
D = 128
NLOOK = 5
NC, NS = 2, 16
NW = NC * NS
BPW = B // NW
CHUNK_ROWS = 2
IDX_PER_CHUNK = CHUNK_ROWS * L
NCHUNK = BPW // CHUNK_ROWS
DCH = D // 16


def _sc_pool_body(codes_hbm, diag_t, proc_t, med_t, out_hbm,
                  idx_v, rows0_v, rows1_v, acc_v, sem0, sem1):
    wid = lax.axis_index("s") * NC + lax.axis_index("c")
    base = wid * BPW
    tables = (diag_t, proc_t, diag_t, proc_t, med_t)
    bufs = ((rows0_v, sem0), (rows1_v, sem1))

    for look in range(NLOOK):
        table = tables[look]
        pltpu.sync_copy(codes_hbm.at[look, wid], idx_v)

        def gather(g, buf, sem, table=table):
            return pltpu.make_async_copy(table.at[idx_v.at[g]], buf, sem)

        def reduce_store(g, buf):
            def red(j, accs):
                a0 = tuple(accs[d] + buf[j, pl.ds(d * 16, 16)]
                           for d in range(DCH))
                a1 = tuple(accs[DCH + d] + buf[L + j, pl.ds(d * 16, 16)]
                           for d in range(DCH))
                return a0 + a1

            zero = jnp.zeros((16,), jnp.float32)
            accs = lax.fori_loop(0, L, red, (zero,) * (2 * DCH))
            for d in range(DCH):
                acc_v[2 * g, pl.ds(d * 16, 16)] = accs[d]
                acc_v[2 * g + 1, pl.ds(d * 16, 16)] = accs[DCH + d]

        gather(0, *bufs[0]).start()

        def pair_body(gg, carry):
            g0 = 2 * gg
            gather(g0 + 1, *bufs[1]).start()
            gather(g0, *bufs[0]).wait()
            reduce_store(g0, bufs[0][0])

            @pl.when(g0 + 2 < NCHUNK)
            def _():
                gather(g0 + 2, *bufs[0]).start()

            gather(g0 + 1, *bufs[1]).wait()
            reduce_store(g0 + 1, bufs[1][0])
            return carry

        lax.fori_loop(0, NCHUNK // 2, pair_body, 0)
        pltpu.sync_copy(acc_v, out_hbm.at[look, pl.ds(base, BPW)])


def _sc_pool(codes, diag_table, proc_table, med_table):
    run = functools.partial(
        pl.kernel,
        mesh=plsc.VectorSubcoreMesh(core_axis_name="c", subcore_axis_name="s"),
        out_type=jax.ShapeDtypeStruct((NLOOK, B, D), jnp.float32),
        scratch_types=[
            pltpu.VMEM((NCHUNK, IDX_PER_CHUNK), jnp.int32),
            pltpu.VMEM((IDX_PER_CHUNK, D), jnp.float32),
            pltpu.VMEM((IDX_PER_CHUNK, D), jnp.float32),
            pltpu.VMEM((BPW, D), jnp.float32),
            pltpu.SemaphoreType.DMA,
            pltpu.SemaphoreType.DMA,
        ],
    )(_sc_pool_body)
    return run(codes, diag_table, proc_table, med_table)


OUT_RAW = 1000
OUT_PAD = 1024
BLK = 128


def _mlp_body(ed, ep, epd, epp, em, w1a, w1b, b1r, w2a, w2b, w2c, b2r,
              w3t, b3r, out):
    cur = ed[:] @ w1a[:] + ep[:] @ w1b[:] + b1r[:]
    prev = epd[:] @ w1a[:] + epp[:] @ w1b[:] + b1r[:]
    h = jnp.maximum(
        cur @ w2a[:] + prev @ w2b[:] + em[:] @ w2c[:] + b2r[:], 0.0)
    out[:] = jax.nn.sigmoid(h @ w3t[:] + b3r[:])


def _mlp(ed, ep, epd, epp, em, w1a, w1b, b1r, w2a, w2b, w2c, b2r, w3t, b3r):
    full = lambda shape: pl.BlockSpec(shape, lambda i: (0, 0))
    blk = pl.BlockSpec((BLK, D), lambda i: (i, 0))
    return pl.pallas_call(
        _mlp_body,
        grid=(B // BLK,),
        in_specs=[blk, blk, blk, blk, blk,
                  full((D, D)), full((D, D)), full((1, D)),
                  full((D, 256)), full((D, 256)), full((D, 256)),
                  full((1, 256)),
                  full((256, OUT_PAD)), full((1, OUT_PAD))],
        out_specs=pl.BlockSpec((BLK, OUT_PAD), lambda i: (i, 0)),
        out_shape=jax.ShapeDtypeStruct((B, OUT_PAD), jnp.float32),
    )(ed, ep, epd, epp, em, w1a, w1b, b1r, w2a, w2b, w2c, b2r, w3t, b3r)


def kernel(diag_codes, proc_codes, prev_diag_codes, prev_proc_codes,
           prev_med_codes, diag_table, proc_table, med_table,
           W1, b1, W2, b2, W3, b3):
    codes = jnp.stack([diag_codes, proc_codes, prev_diag_codes,
                       prev_proc_codes, prev_med_codes])
    codes = codes.astype(jnp.int32).reshape(NLOOK, NW, NCHUNK, IDX_PER_CHUNK)
    pooled = _sc_pool(codes, diag_table, proc_table, med_table)

    w1a = W1[:, :D].T
    w1b = W1[:, D:].T
    w2a = W2[:, :D].T
    w2b = W2[:, D:2 * D].T
    w2c = W2[:, 2 * D:].T
    w3t = jnp.zeros((256, OUT_PAD), jnp.float32).at[:, :OUT_RAW].set(W3.T)
    b3r = jnp.zeros((1, OUT_PAD), jnp.float32).at[0, :OUT_RAW].set(b3)
    out = _mlp(pooled[0], pooled[1], pooled[2], pooled[3], pooled[4],
               w1a, w1b, b1.reshape(1, D), w2a, w2b, w2c, b2.reshape(1, 256),
               w3t, b3r)
    return out[:, :OUT_RAW]


# speedup vs baseline: 3.9325x; 1.0372x over previous
import functools

import jax
import jax.numpy as jnp
from jax import lax
from jax.experimental import pallas as pl
from jax.experimental.pallas import tpu as pltpu
from jax.experimental.pallas import tpu_sc as plsc

B = 1024
L = 50
D = 128
NLOOK = 5
NC, NS = 2, 16
NW = NC * NS
BPW = B // NW
CHUNK_ROWS = 2
IDX_PER_CHUNK = CHUNK_ROWS * L
NCHUNK = BPW // CHUNK_ROWS
DCH = D // 16


def _sc_pool_body(codes_hbm, diag_t, proc_t, med_t, out_hbm,
                  idx_v, rows0_v, rows1_v, acc_v,
                  sem0, sem1, sem_idx, sem_out):
    wid = lax.axis_index("s") * NC + lax.axis_index("c")
    base = wid * BPW
    tables = (diag_t, proc_t, diag_t, proc_t, med_t)
    bufs = ((rows0_v, sem0), (rows1_v, sem1))

    def idx_copy(look, slot):
        return pltpu.make_async_copy(codes_hbm.at[look, wid],
                                     idx_v.at[slot], sem_idx)

    def out_copy(look, slot):
        return pltpu.make_async_copy(acc_v.at[slot],
                                     out_hbm.at[look, pl.ds(base, BPW)],
                                     sem_out)

    idx_copy(0, 0).start()
    for look in range(NLOOK):
        slot = look % 2
        table = tables[look]
        idx_copy(look, slot).wait()
        if look + 1 < NLOOK:
            idx_copy(look + 1, 1 - slot).start()
        if look >= 2:
            out_copy(look - 2, slot).wait()

        def gather(g, buf, sem, table=table, slot=slot):
            return pltpu.make_async_copy(table.at[idx_v.at[slot, g]],
                                         buf, sem)

        def reduce_store(g, buf, slot=slot):
            def red(j, accs):
                a0 = tuple(accs[d] + buf[j, pl.ds(d * 16, 16)]
                           for d in range(DCH))
                a1 = tuple(accs[DCH + d] + buf[L + j, pl.ds(d * 16, 16)]
                           for d in range(DCH))
                return a0 + a1

            zero = jnp.zeros((16,), jnp.float32)
            accs = lax.fori_loop(0, L, red, (zero,) * (2 * DCH))
            for d in range(DCH):
                acc_v[slot, 2 * g, pl.ds(d * 16, 16)] = accs[d]
                acc_v[slot, 2 * g + 1, pl.ds(d * 16, 16)] = accs[DCH + d]

        gather(0, *bufs[0]).start()

        def pair_body(gg, carry):
            g0 = 2 * gg
            gather(g0 + 1, *bufs[1]).start()
            gather(g0, *bufs[0]).wait()
            reduce_store(g0, bufs[0][0])

            @pl.when(g0 + 2 < NCHUNK)
            def _():
                gather(g0 + 2, *bufs[0]).start()

            gather(g0 + 1, *bufs[1]).wait()
            reduce_store(g0 + 1, bufs[1][0])
            return carry

        lax.fori_loop(0, NCHUNK // 2, pair_body, 0)
        out_copy(look, slot).start()

    out_copy(NLOOK - 2, (NLOOK - 2) % 2).wait()
    out_copy(NLOOK - 1, (NLOOK - 1) % 2).wait()


def _sc_pool(codes, diag_table, proc_table, med_table):
    run = functools.partial(
        pl.kernel,
        mesh=plsc.VectorSubcoreMesh(core_axis_name="c", subcore_axis_name="s"),
        out_type=jax.ShapeDtypeStruct((NLOOK, B, D), jnp.float32),
        scratch_types=[
            pltpu.VMEM((2, NCHUNK, IDX_PER_CHUNK), jnp.int32),
            pltpu.VMEM((IDX_PER_CHUNK, D), jnp.float32),
            pltpu.VMEM((IDX_PER_CHUNK, D), jnp.float32),
            pltpu.VMEM((2, BPW, D), jnp.float32),
            pltpu.SemaphoreType.DMA,
            pltpu.SemaphoreType.DMA,
            pltpu.SemaphoreType.DMA,
            pltpu.SemaphoreType.DMA,
        ],
    )(_sc_pool_body)
    return run(codes, diag_table, proc_table, med_table)


OUT_RAW = 1000
OUT_PAD = 1024
BLK = 128


def _mlp_body(ed, ep, epd, epp, em, w1a, w1b, b1r, w2a, w2b, w2c, b2r,
              w3t, b3r, out):
    cur = ed[:] @ w1a[:] + ep[:] @ w1b[:] + b1r[:]
    prev = epd[:] @ w1a[:] + epp[:] @ w1b[:] + b1r[:]
    h = jnp.maximum(
        cur @ w2a[:] + prev @ w2b[:] + em[:] @ w2c[:] + b2r[:], 0.0)
    out[:] = jax.nn.sigmoid(h @ w3t[:] + b3r[:])


def _mlp(ed, ep, epd, epp, em, w1a, w1b, b1r, w2a, w2b, w2c, b2r, w3t, b3r):
    full = lambda shape: pl.BlockSpec(shape, lambda i: (0, 0))
    blk = pl.BlockSpec((BLK, D), lambda i: (i, 0))
    return pl.pallas_call(
        _mlp_body,
        grid=(B // BLK,),
        in_specs=[blk, blk, blk, blk, blk,
                  full((D, D)), full((D, D)), full((1, D)),
                  full((D, 256)), full((D, 256)), full((D, 256)),
                  full((1, 256)),
                  full((256, OUT_PAD)), full((1, OUT_PAD))],
        out_specs=pl.BlockSpec((BLK, OUT_PAD), lambda i: (i, 0)),
        out_shape=jax.ShapeDtypeStruct((B, OUT_PAD), jnp.float32),
    )(ed, ep, epd, epp, em, w1a, w1b, b1r, w2a, w2b, w2c, b2r, w3t, b3r)


def kernel(diag_codes, proc_codes, prev_diag_codes, prev_proc_codes,
           prev_med_codes, diag_table, proc_table, med_table,
           W1, b1, W2, b2, W3, b3):
    codes = jnp.stack([diag_codes, proc_codes, prev_diag_codes,
                       prev_proc_codes, prev_med_codes])
    codes = codes.astype(jnp.int32).reshape(NLOOK, NW, NCHUNK, IDX_PER_CHUNK)
    pooled = _sc_pool(codes, diag_table, proc_table, med_table)

    w1a = W1[:, :D].T
    w1b = W1[:, D:].T
    w2a = W2[:, :D].T
    w2b = W2[:, D:2 * D].T
    w2c = W2[:, 2 * D:].T
    w3t = jnp.zeros((256, OUT_PAD), jnp.float32).at[:, :OUT_RAW].set(W3.T)
    b3r = jnp.zeros((1, OUT_PAD), jnp.float32).at[0, :OUT_RAW].set(b3)
    out = _mlp(pooled[0], pooled[1], pooled[2], pooled[3], pooled[4],
               w1a, w1b, b1.reshape(1, D), w2a, w2b, w2c, b2.reshape(1, 256),
               w3t, b3r)
    return out[:, :OUT_RAW]


# speedup vs baseline: 3.9693x; 1.0094x over previous
import functools

import jax
import jax.numpy as jnp
from jax import lax
from jax.experimental import pallas as pl
from jax.experimental.pallas import tpu as pltpu
from jax.experimental.pallas import tpu_sc as plsc

B = 1024
L = 50
D = 128
NLOOK = 5
NC, NS = 2, 16
NW = NC * NS
BPW = B // NW
CHUNK_ROWS = 2
IDX_PER_CHUNK = CHUNK_ROWS * L
NCHUNK = BPW // CHUNK_ROWS
DCH = D // 16


def _sc_pool_body(codes_hbm, diag_t, proc_t, med_t, out_hbm,
                  idx_v, rows0_v, rows1_v, acc_v,
                  sem0, sem1, sem_idx, sem_out):
    wid = lax.axis_index("s") * NC + lax.axis_index("c")
    base = wid * BPW
    tables = (diag_t, proc_t, diag_t, proc_t, med_t)
    bufs = ((rows0_v, sem0), (rows1_v, sem1))

    def idx_copy(look, slot):
        return pltpu.make_async_copy(codes_hbm.at[look, wid],
                                     idx_v.at[slot], sem_idx)

    def out_copy(look, slot):
        return pltpu.make_async_copy(acc_v.at[slot],
                                     out_hbm.at[look, pl.ds(base, BPW)],
                                     sem_out)

    idx_copy(0, 0).start()
    for look in range(NLOOK):
        slot = look % 2
        table = tables[look]
        idx_copy(look, slot).wait()
        if look + 1 < NLOOK:
            idx_copy(look + 1, 1 - slot).start()
        if look >= 2:
            out_copy(look - 2, slot).wait()

        def gather(g, buf, sem, table=table, slot=slot):
            return pltpu.make_async_copy(table.at[idx_v.at[slot, g]],
                                         buf, sem)

        def reduce_store(g, buf, slot=slot):
            def red(j, accs):
                a0 = tuple(accs[d] + buf[j, pl.ds(d * 16, 16)]
                           for d in range(DCH))
                a1 = tuple(accs[DCH + d] + buf[L + j, pl.ds(d * 16, 16)]
                           for d in range(DCH))
                return a0 + a1

            zero = jnp.zeros((16,), jnp.float32)
            accs = lax.fori_loop(0, L, red, (zero,) * (2 * DCH))
            for d in range(DCH):
                acc_v[slot, 2 * g, pl.ds(d * 16, 16)] = accs[d]
                acc_v[slot, 2 * g + 1, pl.ds(d * 16, 16)] = accs[DCH + d]

        gather(0, *bufs[0]).start()

        def pair_body(gg, carry):
            g0 = 2 * gg
            gather(g0 + 1, *bufs[1]).start()
            gather(g0, *bufs[0]).wait()
            reduce_store(g0, bufs[0][0])

            @pl.when(g0 + 2 < NCHUNK)
            def _():
                gather(g0 + 2, *bufs[0]).start()

            gather(g0 + 1, *bufs[1]).wait()
            reduce_store(g0 + 1, bufs[1][0])
            return carry

        lax.fori_loop(0, NCHUNK // 2, pair_body, 0)
        out_copy(look, slot).start()

    out_copy(NLOOK - 2, (NLOOK - 2) % 2).wait()
    out_copy(NLOOK - 1, (NLOOK - 1) % 2).wait()


def _sc_pool(codes, diag_table, proc_table, med_table):
    run = functools.partial(
        pl.kernel,
        mesh=plsc.VectorSubcoreMesh(core_axis_name="c", subcore_axis_name="s"),
        out_type=jax.ShapeDtypeStruct((NLOOK, B, D), jnp.float32),
        scratch_types=[
            pltpu.VMEM((2, NCHUNK, IDX_PER_CHUNK), jnp.int32),
            pltpu.VMEM((IDX_PER_CHUNK, D), jnp.float32),
            pltpu.VMEM((IDX_PER_CHUNK, D), jnp.float32),
            pltpu.VMEM((2, BPW, D), jnp.float32),
            pltpu.SemaphoreType.DMA,
            pltpu.SemaphoreType.DMA,
            pltpu.SemaphoreType.DMA,
            pltpu.SemaphoreType.DMA,
        ],
    )(_sc_pool_body)
    return run(codes, diag_table, proc_table, med_table)


OUT_RAW = 1000
OUT_PAD = 1024
BLK = 128


def _mlp_body(ed, ep, epd, epp, em, w1a, w1b, b1r, w2a, w2b, w2c, b2r,
              w3t, b3r, out):
    cur = ed[:] @ w1a[:] + ep[:] @ w1b[:] + b1r[:]
    prev = epd[:] @ w1a[:] + epp[:] @ w1b[:] + b1r[:]
    h = jnp.maximum(
        cur @ w2a[:] + prev @ w2b[:] + em[:] @ w2c[:] + b2r[:], 0.0)
    out[:] = jax.nn.sigmoid(h @ w3t[:] + b3r[:])


def _mlp(ed, ep, epd, epp, em, w1a, w1b, b1r, w2a, w2b, w2c, b2r, w3t, b3r):
    return pl.pallas_call(
        _mlp_body,
        out_shape=jax.ShapeDtypeStruct((B, OUT_PAD), jnp.float32),
    )(ed, ep, epd, epp, em, w1a, w1b, b1r, w2a, w2b, w2c, b2r, w3t, b3r)


def kernel(diag_codes, proc_codes, prev_diag_codes, prev_proc_codes,
           prev_med_codes, diag_table, proc_table, med_table,
           W1, b1, W2, b2, W3, b3):
    codes = jnp.stack([diag_codes, proc_codes, prev_diag_codes,
                       prev_proc_codes, prev_med_codes])
    codes = codes.astype(jnp.int32).reshape(NLOOK, NW, NCHUNK, IDX_PER_CHUNK)
    pooled = _sc_pool(codes, diag_table, proc_table, med_table)

    w1a = W1[:, :D].T
    w1b = W1[:, D:].T
    w2a = W2[:, :D].T
    w2b = W2[:, D:2 * D].T
    w2c = W2[:, 2 * D:].T
    w3t = jnp.zeros((256, OUT_PAD), jnp.float32).at[:, :OUT_RAW].set(W3.T)
    b3r = jnp.zeros((1, OUT_PAD), jnp.float32).at[0, :OUT_RAW].set(b3)
    out = _mlp(pooled[0], pooled[1], pooled[2], pooled[3], pooled[4],
               w1a, w1b, b1.reshape(1, D), w2a, w2b, w2c, b2.reshape(1, 256),
               w3t, b3r)
    return out[:, :OUT_RAW]
